# Initial kernel scaffold; baseline (speedup 1.0000x reference)
import jax
import jax.numpy as jnp
from jax.experimental import pallas as pl


def kernel(ycbcr, W, b, gamma, beta):
    raise NotImplementedError("write your pallas kernel here")


# speedup vs baseline: 40.0052x; 40.0052x over previous
import functools

import jax
import jax.numpy as jnp
from jax import lax
from jax.experimental import pallas as pl
from jax.experimental.pallas import tpu as pltpu
from jax.experimental.pallas import tpu_sc as plsc

NUM_BINS = 64
LANES = 16
BIN_SLOTS = NUM_BINS * LANES
CHUNK_ROWS = 64
N_CHUNKS = 512 // CHUNK_ROWS
HIGH = jax.lax.Precision.HIGHEST


def _sc_hist_body(ycbcr_hbm, out_hbm, buf0, buf1, bins, sem0, sem1):
    b_img = lax.axis_index("s") * 2 + lax.axis_index("c")
    bufs = (buf0, buf1)
    sems = (sem0, sem1)
    lane = lax.iota(jnp.int32, 16)
    ones = jnp.ones((16,), jnp.float32)
    zeros16 = jnp.zeros((16,), jnp.float32)

    def zero_bins():
        def zb(z, c):
            bins[pl.ds(z * 16, 16)] = zeros16
            return c

        lax.fori_loop(0, NUM_BINS, zb, 0)

    def start(ci):
        ch = 1 + ci // N_CHUNKS
        k = ci % N_CHUNKS
        src = ycbcr_hbm.at[b_img, ch, pl.ds(k * CHUNK_ROWS, CHUNK_ROWS), :]
        return pltpu.async_copy(src, bufs[ci % 2], sems[ci % 2])

    def process(buf):
        def pb(i, c):
            v = buf[i // 32, pl.ds((i % 32) * 16, 16)]
            idx = (v * 0.25).astype(jnp.int32)
            idx = jnp.minimum(idx, NUM_BINS - 1)
            idx = jnp.maximum(idx, 0)
            valid = (v >= 0.0) & (v <= 256.0)
            flat = (idx << 4) + lane
            plsc.addupdate_scatter(bins, [flat], ones, mask=valid)
            return c

        lax.fori_loop(0, CHUNK_ROWS * 32, pb, 0)

    total = 2 * N_CHUNKS
    desc = start(0)
    zero_bins()
    for ci in range(total):
        nxt = start(ci + 1) if ci + 1 < total else None
        desc.wait()
        process(bufs[ci % 2])
        if ci % N_CHUNKS == N_CHUNKS - 1:
            ch = 1 + ci // N_CHUNKS
            row = b_img + 32 * (ch - 1)
            pltpu.sync_copy(bins, out_hbm.at[row])
            if ci + 1 < total:
                zero_bins()
        desc = nxt


def _make_sc_hist():
    return pl.kernel(
        _sc_hist_body,
        out_type=jax.ShapeDtypeStruct((2 * 32, BIN_SLOTS), jnp.float32),
        mesh=plsc.VectorSubcoreMesh(
            core_axis_name="c", subcore_axis_name="s", num_cores=2, num_subcores=16
        ),
        compiler_params=pltpu.CompilerParams(needs_layout_passes=False),
        scratch_types=[
            pltpu.VMEM((CHUNK_ROWS, 512), jnp.float32),
            pltpu.VMEM((CHUNK_ROWS, 512), jnp.float32),
            pltpu.VMEM((BIN_SLOTS,), jnp.float32),
            pltpu.SemaphoreType.DMA,
            pltpu.SemaphoreType.DMA,
        ],
    )


def _stats_body(x_ref, o_ref):
    x = x_ref[0, 0]
    n = 512 * 512
    mean = jnp.sum(x) * (1.0 / n)
    d = x - mean
    gv = jnp.sum(d * d) * (1.0 / (n - 1))

    r64 = lax.broadcasted_iota(jnp.int32, (64, 512), 0)
    c512 = lax.broadcasted_iota(jnp.int32, (64, 512), 1)
    A = jnp.where((c512 >> 3) == r64, 1.0, 0.0).astype(jnp.float32)
    r512 = lax.broadcasted_iota(jnp.int32, (512, 64), 0)
    c64 = lax.broadcasted_iota(jnp.int32, (512, 64), 1)
    AT = jnp.where((r512 >> 3) == c64, 1.0, 0.0).astype(jnp.float32)

    rowsum = jnp.dot(A, x, precision=HIGH, preferred_element_type=jnp.float32)
    psums = jnp.dot(rowsum, AT, precision=HIGH, preferred_element_type=jnp.float32)
    pmeans = psums * (1.0 / 64.0)
    pm_rows = jnp.dot(AT, pmeans, precision=HIGH, preferred_element_type=jnp.float32)
    pm_full = jnp.dot(pm_rows, A, precision=HIGH, preferred_element_type=jnp.float32)
    dl = x - pm_full
    lv = jnp.sum(dl * dl) * (1.0 / (63.0 * 64.0 * 64.0))

    ii = lax.broadcasted_iota(jnp.int32, (8, 128), 1)
    jj = lax.broadcasted_iota(jnp.int32, (8, 128), 0)
    first = jj == 0
    o_ref[0, 0] = jnp.where(
        first & (ii == 0), gv, jnp.where(first & (ii == 1), lv, 0.0)
    )


def _proj_body(counts_ref, stats_ref, wt_ref, b_ref, g_ref, beta_ref, o_ref):
    counts = counts_ref[...]
    jbin = lax.broadcasted_iota(jnp.int32, (BIN_SLOTS, NUM_BINS), 0)
    bb = lax.broadcasted_iota(jnp.int32, (BIN_SLOTS, NUM_BINS), 1)
    sel = jnp.where((jbin >> 4) == bb, 1.0, 0.0).astype(jnp.float32)
    counts_b = jnp.dot(counts, sel, precision=HIGH, preferred_element_type=jnp.float32)
    tot = jnp.sum(counts_b, axis=1, keepdims=True)
    hist = counts_b / (tot + 1e-8)

    cb = hist[0:32, :]
    cr = hist[32:64, :]
    s_cb = stats_ref[0, :, 0, :]
    s_cr = stats_ref[1, :, 0, :]
    feats = jnp.concatenate(
        [
            cb,
            cr,
            s_cb[:, 0:1],
            s_cb[:, 1:2],
            s_cr[:, 0:1],
            s_cr[:, 1:2],
            jnp.zeros((32, 124), jnp.float32),
        ],
        axis=1,
    )

    x = jnp.dot(feats, wt_ref[...], preferred_element_type=jnp.float32)
    x = x + b_ref[...]
    x = jnp.maximum(x, 0.0)
    m = jnp.mean(x, axis=0, keepdims=True)
    var = jnp.mean((x - m) * (x - m), axis=0, keepdims=True)
    y = (x - m) * lax.rsqrt(var + 1e-5)
    o_ref[...] = g_ref[...] * y + beta_ref[...]


def kernel(ycbcr, W, b, gamma, beta):
    counts = _make_sc_hist()(ycbcr)

    stats = pl.pallas_call(
        _stats_body,
        grid=(2, 32),
        in_specs=[pl.BlockSpec((1, 1, 512, 512), lambda c, b_: (b_, c + 1, 0, 0))],
        out_specs=pl.BlockSpec((1, 1, 8, 128), lambda c, b_: (c, b_, 0, 0)),
        out_shape=jax.ShapeDtypeStruct((2, 32, 8, 128), jnp.float32),
    )(ycbcr)

    wt_pad = jnp.zeros((256, 256), jnp.float32).at[0:132, :].set(W.T)
    out = pl.pallas_call(
        _proj_body,
        out_shape=jax.ShapeDtypeStruct((32, 256), jnp.float32),
    )(
        counts,
        stats,
        wt_pad,
        b.reshape(1, 256),
        gamma.reshape(1, 256),
        beta.reshape(1, 256),
    )
    return out


# speedup vs baseline: 87.5573x; 2.1886x over previous
import functools

import jax
import jax.numpy as jnp
from jax import lax
from jax.experimental import pallas as pl
from jax.experimental.pallas import tpu as pltpu
from jax.experimental.pallas import tpu_sc as plsc

NUM_BINS = 64
LANES = 16
BIN_SLOTS = NUM_BINS * LANES
CHUNK_ROWS = 64
N_CHUNKS = 512 // CHUNK_ROWS
HIGH = jax.lax.Precision.HIGHEST


def _sc_hist_body(ycbcr_hbm, out_hbm, buf0, buf1, bins, sem0, sem1):
    b_img = lax.axis_index("s") * 2 + lax.axis_index("c")
    bufs = (buf0, buf1)
    sems = (sem0, sem1)
    lane = lax.iota(jnp.int32, 16)
    ones = jnp.ones((16,), jnp.float32)
    zeros16 = jnp.zeros((16,), jnp.float32)

    def zero_bins():
        def zb(z, c):
            bins[pl.ds(z * 16, 16)] = zeros16
            return c

        lax.fori_loop(0, NUM_BINS, zb, 0)

    def start(ci):
        ch = 1 + ci // N_CHUNKS
        k = ci % N_CHUNKS
        src = ycbcr_hbm.at[b_img, ch, pl.ds(k * CHUNK_ROWS, CHUNK_ROWS), :]
        return pltpu.async_copy(src, bufs[ci % 2], sems[ci % 2])

    def process(buf):
        @plsc.parallel_loop(0, CHUNK_ROWS * 32, 1, unroll=8)
        def pb(i):
            v = buf[i >> 5, pl.ds((i & 31) << 4, 16)]
            idx = (v * 0.25).astype(jnp.int32)
            idx = jnp.minimum(idx, NUM_BINS - 1)
            idx = jnp.maximum(idx, 0)
            valid = (v >= 0.0) & (v <= 256.0)
            flat = (idx << 4) + lane
            plsc.addupdate_scatter(bins, [flat], ones, mask=valid)

    total = 2 * N_CHUNKS
    desc = start(0)
    zero_bins()
    for ci in range(total):
        nxt = start(ci + 1) if ci + 1 < total else None
        desc.wait()
        process(bufs[ci % 2])
        if ci % N_CHUNKS == N_CHUNKS - 1:
            ch = 1 + ci // N_CHUNKS
            row = b_img + 32 * (ch - 1)
            pltpu.sync_copy(bins, out_hbm.at[row])
            if ci + 1 < total:
                zero_bins()
        desc = nxt


def _make_sc_hist():
    return pl.kernel(
        _sc_hist_body,
        out_type=jax.ShapeDtypeStruct((2 * 32, BIN_SLOTS), jnp.float32),
        mesh=plsc.VectorSubcoreMesh(
            core_axis_name="c", subcore_axis_name="s", num_cores=2, num_subcores=16
        ),
        compiler_params=pltpu.CompilerParams(needs_layout_passes=False),
        scratch_types=[
            pltpu.VMEM((CHUNK_ROWS, 512), jnp.float32),
            pltpu.VMEM((CHUNK_ROWS, 512), jnp.float32),
            pltpu.VMEM((BIN_SLOTS,), jnp.float32),
            pltpu.SemaphoreType.DMA,
            pltpu.SemaphoreType.DMA,
        ],
    )


def _stats_body(x_ref, o_ref):
    x = x_ref[0, 0]
    n = 512 * 512
    mean = jnp.sum(x) * (1.0 / n)
    d = x - mean
    gv = jnp.sum(d * d) * (1.0 / (n - 1))

    r64 = lax.broadcasted_iota(jnp.int32, (64, 512), 0)
    c512 = lax.broadcasted_iota(jnp.int32, (64, 512), 1)
    A = jnp.where((c512 >> 3) == r64, 1.0, 0.0).astype(jnp.float32)
    r512 = lax.broadcasted_iota(jnp.int32, (512, 64), 0)
    c64 = lax.broadcasted_iota(jnp.int32, (512, 64), 1)
    AT = jnp.where((r512 >> 3) == c64, 1.0, 0.0).astype(jnp.float32)

    rowsum = jnp.dot(A, x, precision=HIGH, preferred_element_type=jnp.float32)
    psums = jnp.dot(rowsum, AT, precision=HIGH, preferred_element_type=jnp.float32)
    pmeans = psums * (1.0 / 64.0)
    pm_rows = jnp.dot(AT, pmeans, precision=HIGH, preferred_element_type=jnp.float32)
    pm_full = jnp.dot(pm_rows, A, precision=HIGH, preferred_element_type=jnp.float32)
    dl = x - pm_full
    lv = jnp.sum(dl * dl) * (1.0 / (63.0 * 64.0 * 64.0))

    ii = lax.broadcasted_iota(jnp.int32, (8, 128), 1)
    jj = lax.broadcasted_iota(jnp.int32, (8, 128), 0)
    first = jj == 0
    o_ref[0, 0] = jnp.where(
        first & (ii == 0), gv, jnp.where(first & (ii == 1), lv, 0.0)
    )


def _proj_body(counts_ref, stats_ref, wt_ref, b_ref, g_ref, beta_ref, o_ref):
    counts = counts_ref[...]
    jbin = lax.broadcasted_iota(jnp.int32, (BIN_SLOTS, NUM_BINS), 0)
    bb = lax.broadcasted_iota(jnp.int32, (BIN_SLOTS, NUM_BINS), 1)
    sel = jnp.where((jbin >> 4) == bb, 1.0, 0.0).astype(jnp.float32)
    counts_b = jnp.dot(counts, sel, precision=HIGH, preferred_element_type=jnp.float32)
    tot = jnp.sum(counts_b, axis=1, keepdims=True)
    hist = counts_b / (tot + 1e-8)

    cb = hist[0:32, :]
    cr = hist[32:64, :]
    s_cb = stats_ref[0, :, 0, :]
    s_cr = stats_ref[1, :, 0, :]
    feats = jnp.concatenate(
        [
            cb,
            cr,
            s_cb[:, 0:1],
            s_cb[:, 1:2],
            s_cr[:, 0:1],
            s_cr[:, 1:2],
            jnp.zeros((32, 124), jnp.float32),
        ],
        axis=1,
    )

    x = jnp.dot(feats, wt_ref[...], preferred_element_type=jnp.float32)
    x = x + b_ref[...]
    x = jnp.maximum(x, 0.0)
    m = jnp.mean(x, axis=0, keepdims=True)
    var = jnp.mean((x - m) * (x - m), axis=0, keepdims=True)
    y = (x - m) * lax.rsqrt(var + 1e-5)
    o_ref[...] = g_ref[...] * y + beta_ref[...]


def kernel(ycbcr, W, b, gamma, beta):
    counts = _make_sc_hist()(ycbcr)

    stats = pl.pallas_call(
        _stats_body,
        grid=(2, 32),
        in_specs=[pl.BlockSpec((1, 1, 512, 512), lambda c, b_: (b_, c + 1, 0, 0))],
        out_specs=pl.BlockSpec((1, 1, 8, 128), lambda c, b_: (c, b_, 0, 0)),
        out_shape=jax.ShapeDtypeStruct((2, 32, 8, 128), jnp.float32),
    )(ycbcr)

    wt_pad = jnp.zeros((256, 256), jnp.float32).at[0:132, :].set(W.T)
    out = pl.pallas_call(
        _proj_body,
        out_shape=jax.ShapeDtypeStruct((32, 256), jnp.float32),
    )(
        counts,
        stats,
        wt_pad,
        b.reshape(1, 256),
        gamma.reshape(1, 256),
        beta.reshape(1, 256),
    )
    return out
